# baseline (speedup 1.0000x reference)
import jax
import jax.numpy as jnp
from jax.experimental import pallas as pl
from jax.experimental.pallas import tpu as pltpu

_OUT_ACTIONS = 18
_CHUNK = 2048
_NBUF = 4
_BLOCK_B = 4096


def _mlp_block(x_f32, w1, b1, w2, b2):
    x = x_f32.astype(jnp.bfloat16)
    h = jnp.dot(x, w1, preferred_element_type=jnp.float32)
    h = jnp.maximum(h + b1, 0.0).astype(jnp.bfloat16)
    y = jnp.dot(h, w2, preferred_element_type=jnp.float32)
    return (y + b2)[:, :_OUT_ACTIONS]


def _ring_kernel(x_any, w1_ref, b1_ref, w2_ref, b2_ref, o_ref, xbuf, sems,
                 *, nsteps):
    w1 = w1_ref[...].astype(jnp.bfloat16)
    w2 = w2_ref[...].astype(jnp.bfloat16)
    b1 = b1_ref[...]
    b2 = b2_ref[...]

    def dma(step, slot):
        return pltpu.make_async_copy(
            x_any.at[pl.ds(step * _CHUNK, _CHUNK), :],
            xbuf.at[slot], sems.at[slot])

    for s in range(min(_NBUF, nsteps)):
        dma(s, s).start(priority=s % 2)
    for step in range(nsteps):
        slot = step % _NBUF
        dma(step, slot).wait()
        y = _mlp_block(xbuf[slot], w1, b1, w2, b2)
        o_ref[pl.ds(step * _CHUNK, _CHUNK), :] = y
        nxt = step + _NBUF
        if nxt < nsteps:
            dma(nxt, slot).start(priority=nxt % 2)


def _grid_kernel(x_ref, w1_ref, b1_ref, w2_ref, b2_ref, o_ref):
    o_ref[...] = _mlp_block(
        x_ref[...], w1_ref[...].astype(jnp.bfloat16), b1_ref[...],
        w2_ref[...].astype(jnp.bfloat16), b2_ref[...])


@jax.jit
def kernel(x, w1, b1, w2, b2):
    B, K = x.shape
    Hp = w1.shape[1]
    Np = w2.shape[1]
    flops = 2 * B * (K * Hp + Hp * Np)
    w_bytes = (w1.size + b1.size + w2.size + b2.size) * 4
    cost = pl.CostEstimate(
        flops=flops, transcendentals=0,
        bytes_accessed=B * K * 4 + w_bytes + B * _OUT_ACTIONS * 4)

    if B % _CHUNK == 0:
        import functools
        nsteps = B // _CHUNK
        return pl.pallas_call(
            functools.partial(_ring_kernel, nsteps=nsteps),
            out_shape=jax.ShapeDtypeStruct((B, _OUT_ACTIONS), jnp.float32),
            in_specs=[
                pl.BlockSpec(memory_space=pl.ANY),
                pl.BlockSpec(memory_space=pltpu.MemorySpace.VMEM),
                pl.BlockSpec(memory_space=pltpu.MemorySpace.VMEM),
                pl.BlockSpec(memory_space=pltpu.MemorySpace.VMEM),
                pl.BlockSpec(memory_space=pltpu.MemorySpace.VMEM),
            ],
            out_specs=pl.BlockSpec(memory_space=pltpu.MemorySpace.VMEM),
            scratch_shapes=[
                pltpu.VMEM((_NBUF, _CHUNK, K), jnp.float32),
                pltpu.SemaphoreType.DMA((_NBUF,)),
            ],
            cost_estimate=cost,
        )(x, w1, b1, w2, b2)

    block_b = min(_BLOCK_B, B)
    nb = pl.cdiv(B, block_b)
    return pl.pallas_call(
        _grid_kernel,
        out_shape=jax.ShapeDtypeStruct((B, _OUT_ACTIONS), jnp.float32),
        grid=(nb,),
        in_specs=[
            pl.BlockSpec((block_b, K), lambda i: (i, 0)),
            pl.BlockSpec((K, Hp), lambda i: (0, 0)),
            pl.BlockSpec((1, Hp), lambda i: (0, 0)),
            pl.BlockSpec((Hp, Np), lambda i: (0, 0)),
            pl.BlockSpec((1, Np), lambda i: (0, 0)),
        ],
        out_specs=pl.BlockSpec((block_b, _OUT_ACTIONS), lambda i: (i, 0)),
        compiler_params=pltpu.CompilerParams(
            dimension_semantics=("parallel",)),
        cost_estimate=cost,
    )(x, w1, b1, w2, b2)


# speedup vs baseline: 1.0002x; 1.0002x over previous
import jax
import jax.numpy as jnp
from jax.experimental import pallas as pl
from jax.experimental.pallas import tpu as pltpu

_OUT_ACTIONS = 18
_CHUNK = 2048
_NBUF = 4
_BLOCK_B = 4096


def _mlp_block(x_f32, w1, b1, w2, b2):
    x = x_f32.astype(jnp.bfloat16)
    h = jnp.dot(x, w1, preferred_element_type=jnp.float32)
    h = jnp.maximum(h + b1, 0.0).astype(jnp.bfloat16)
    y = jnp.dot(h, w2, preferred_element_type=jnp.float32)
    return (y + b2)[:, :_OUT_ACTIONS]


def _ring_kernel(x_any, w1_ref, b1_ref, w2_ref, b2_ref, o_ref, xbuf, sems,
                 *, nsteps):
    w1 = w1_ref[...].astype(jnp.bfloat16)
    w2 = w2_ref[...].astype(jnp.bfloat16)
    b1 = b1_ref[...]
    b2 = b2_ref[...]

    def dma(step, slot):
        return pltpu.make_async_copy(
            x_any.at[pl.ds(step * _CHUNK, _CHUNK), :],
            xbuf.at[slot], sems.at[slot])

    for s in range(min(_NBUF, nsteps)):
        dma(s, s).start()
    for step in range(nsteps):
        slot = step % _NBUF
        dma(step, slot).wait()
        y = _mlp_block(xbuf[slot], w1, b1, w2, b2)
        o_ref[pl.ds(step * _CHUNK, _CHUNK), :] = y
        nxt = step + _NBUF
        if nxt < nsteps:
            dma(nxt, slot).start()


def _grid_kernel(x_ref, w1_ref, b1_ref, w2_ref, b2_ref, o_ref):
    o_ref[...] = _mlp_block(
        x_ref[...], w1_ref[...].astype(jnp.bfloat16), b1_ref[...],
        w2_ref[...].astype(jnp.bfloat16), b2_ref[...])


@jax.jit
def kernel(x, w1, b1, w2, b2):
    B, K = x.shape
    Hp = w1.shape[1]
    Np = w2.shape[1]
    flops = 2 * B * (K * Hp + Hp * Np)
    w_bytes = (w1.size + b1.size + w2.size + b2.size) * 4
    cost = pl.CostEstimate(
        flops=flops, transcendentals=0,
        bytes_accessed=B * K * 4 + w_bytes + B * _OUT_ACTIONS * 4)

    if B % _CHUNK == 0:
        import functools
        nsteps = B // _CHUNK
        return pl.pallas_call(
            functools.partial(_ring_kernel, nsteps=nsteps),
            out_shape=jax.ShapeDtypeStruct((B, _OUT_ACTIONS), jnp.float32),
            in_specs=[
                pl.BlockSpec(memory_space=pl.ANY),
                pl.BlockSpec(memory_space=pltpu.MemorySpace.VMEM),
                pl.BlockSpec(memory_space=pltpu.MemorySpace.VMEM),
                pl.BlockSpec(memory_space=pltpu.MemorySpace.VMEM),
                pl.BlockSpec(memory_space=pltpu.MemorySpace.VMEM),
            ],
            out_specs=pl.BlockSpec(memory_space=pltpu.MemorySpace.VMEM),
            scratch_shapes=[
                pltpu.VMEM((_NBUF, _CHUNK, K), jnp.float32),
                pltpu.SemaphoreType.DMA((_NBUF,)),
            ],
            cost_estimate=cost,
        )(x, w1, b1, w2, b2)

    block_b = min(_BLOCK_B, B)
    nb = pl.cdiv(B, block_b)
    return pl.pallas_call(
        _grid_kernel,
        out_shape=jax.ShapeDtypeStruct((B, _OUT_ACTIONS), jnp.float32),
        grid=(nb,),
        in_specs=[
            pl.BlockSpec((block_b, K), lambda i: (i, 0)),
            pl.BlockSpec((K, Hp), lambda i: (0, 0)),
            pl.BlockSpec((1, Hp), lambda i: (0, 0)),
            pl.BlockSpec((Hp, Np), lambda i: (0, 0)),
            pl.BlockSpec((1, Np), lambda i: (0, 0)),
        ],
        out_specs=pl.BlockSpec((block_b, _OUT_ACTIONS), lambda i: (i, 0)),
        compiler_params=pltpu.CompilerParams(
            dimension_semantics=("parallel",)),
        cost_estimate=cost,
    )(x, w1, b1, w2, b2)


# speedup vs baseline: 1.1482x; 1.1479x over previous
import jax
import jax.numpy as jnp
from jax.experimental import pallas as pl
from jax.experimental.pallas import tpu as pltpu

_OUT_ACTIONS = 18
_BLOCK_B = 3072


def _mlp_kernel(x_ref, w1_ref, b1_ref, w2_ref, b2_ref, o_ref):
    x = x_ref[...].astype(jnp.bfloat16)
    w1 = w1_ref[...].astype(jnp.bfloat16)
    h = jnp.dot(x, w1, preferred_element_type=jnp.float32)
    h = jnp.maximum(h + b1_ref[...], 0.0).astype(jnp.bfloat16)
    w2 = w2_ref[...].astype(jnp.bfloat16)
    y = jnp.dot(h, w2, preferred_element_type=jnp.float32)
    y = y + b2_ref[...]
    o_ref[...] = y[:, :_OUT_ACTIONS]


@jax.jit
def kernel(x, w1, b1, w2, b2):
    B, K = x.shape
    Hp = w1.shape[1]
    Np = w2.shape[1]
    block_b = min(_BLOCK_B, B)
    nb = pl.cdiv(B, block_b)
    flops = 2 * B * (K * Hp + Hp * Np)
    w_bytes = (w1.size + b1.size + w2.size + b2.size) * 4
    cost = pl.CostEstimate(
        flops=flops, transcendentals=0,
        bytes_accessed=B * K * 4 + w_bytes + B * _OUT_ACTIONS * 4)
    return pl.pallas_call(
        _mlp_kernel,
        out_shape=jax.ShapeDtypeStruct((B, _OUT_ACTIONS), jnp.float32),
        grid=(nb,),
        in_specs=[
            pl.BlockSpec((block_b, K), lambda i: (i, 0)),
            pl.BlockSpec((K, Hp), lambda i: (0, 0)),
            pl.BlockSpec((1, Hp), lambda i: (0, 0)),
            pl.BlockSpec((Hp, Np), lambda i: (0, 0)),
            pl.BlockSpec((1, Np), lambda i: (0, 0)),
        ],
        out_specs=pl.BlockSpec((block_b, _OUT_ACTIONS), lambda i: (i, 0)),
        compiler_params=pltpu.CompilerParams(
            dimension_semantics=("parallel",)),
        cost_estimate=cost,
    )(x, w1, b1, w2, b2)
